# Initial kernel scaffold; baseline (speedup 1.0000x reference)
import jax
import jax.numpy as jnp
from jax.experimental import pallas as pl


def kernel(dom_a_idx, dom_b_idx, table_a, table_b):
    raise NotImplementedError("write your pallas kernel here")


# speedup vs baseline: 2.6033x; 2.6033x over previous
import functools

import jax
import jax.numpy as jnp
from jax import lax
from jax.experimental import pallas as pl
from jax.experimental.pallas import tpu as pltpu
from jax.experimental.pallas import tpu_sc as plsc

_VOCAB_A, _DIM_A = 1000000, 32
_VOCAB_B, _DIM_B = 100000, 64
_BATCH, _HIST = 16384, 50

_L = 128
_TOTAL = _BATCH * _HIST
_ROWS = _TOTAL // _L
_NW = 32
_ROWS_PER_W = _ROWS // _NW
_G = 8
_GROUPS = _ROWS_PER_W // _G


def _body(idx_a_hbm, idx_b_hbm, tab_a_hbm, tab_b_hbm, out_a_hbm, out_b_hbm,
          idx_a_v, idx_b_v, rows_a_v, rows_b_v, sem):
    nc = plsc.get_sparse_core_info().num_cores
    wid = lax.axis_index("s") * nc + lax.axis_index("c")
    base_row = wid * _ROWS_PER_W

    def run(idx_hbm, tab_hbm, out_hbm, idx_v, rows_v):
        def group(g, carry):
            row0 = base_row + g * _G
            pltpu.sync_copy(idx_hbm.at[pl.ds(row0, _G)], idx_v)
            cps = [
                pltpu.async_copy(tab_hbm.at[idx_v.at[j]], rows_v.at[j], sem)
                for j in range(_G)
            ]
            for c in cps:
                c.wait()
            pltpu.sync_copy(rows_v, out_hbm.at[pl.ds(row0, _G)])
            return carry

        lax.fori_loop(0, _GROUPS, group, 0)

    run(idx_a_hbm, tab_a_hbm, out_a_hbm, idx_a_v, rows_a_v)
    run(idx_b_hbm, tab_b_hbm, out_b_hbm, idx_b_v, rows_b_v)


@jax.jit
def _lookup(dom_a_idx, dom_b_idx, table_a, table_b):
    mesh = plsc.VectorSubcoreMesh(core_axis_name="c", subcore_axis_name="s")
    k = pl.kernel(
        _body,
        out_type=(
            jax.ShapeDtypeStruct((_ROWS, _L, _DIM_A), jnp.float32),
            jax.ShapeDtypeStruct((_ROWS, _L, _DIM_B), jnp.float32),
        ),
        mesh=mesh,
        scratch_types=[
            pltpu.VMEM((_G, _L), jnp.int32),
            pltpu.VMEM((_G, _L), jnp.int32),
            pltpu.VMEM((_G, _L, _DIM_A), jnp.float32),
            pltpu.VMEM((_G, _L, _DIM_B), jnp.float32),
            pltpu.SemaphoreType.DMA,
        ],
        compiler_params=pltpu.CompilerParams(use_tc_tiling_on_sc=False),
    )
    ia = dom_a_idx.reshape(_ROWS, _L)
    ib = dom_b_idx.reshape(_ROWS, _L)
    out_a, out_b = k(ia, ib, table_a, table_b)
    return (
        out_a.reshape(_BATCH, _HIST, _DIM_A),
        out_b.reshape(_BATCH, _HIST, _DIM_B),
    )


def kernel(dom_a_idx, dom_b_idx, table_a, table_b):
    return _lookup(dom_a_idx, dom_b_idx, table_a, table_b)


# speedup vs baseline: 2.6476x; 1.0170x over previous
import functools

import jax
import jax.numpy as jnp
from jax import lax
from jax.experimental import pallas as pl
from jax.experimental.pallas import tpu as pltpu
from jax.experimental.pallas import tpu_sc as plsc

_VOCAB_A, _DIM_A = 1000000, 32
_VOCAB_B, _DIM_B = 100000, 64
_BATCH, _HIST = 16384, 50

_L = 128
_TOTAL = _BATCH * _HIST
_ROWS = _TOTAL // _L
_NW = 32
_ROWS_PER_W = _ROWS // _NW
_G = 8
_GROUPS = _ROWS_PER_W // _G


def _body(idx_a_hbm, idx_b_hbm, tab_a_hbm, tab_b_hbm, out_a_hbm, out_b_hbm,
          idx_a_v, idx_b_v, rows_a_v, rows_b_v, sem_g, sem_st):
    nc = plsc.get_sparse_core_info().num_cores
    wid = lax.axis_index("s") * nc + lax.axis_index("c")
    base_row = wid * _ROWS_PER_W

    def run(idx_hbm, tab_hbm, out_hbm, idx_v, rows_v):
        def group(g, carry):
            row0 = base_row + g * _G

            @pl.when(g > 0)
            def _drain_prev():
                for j in range(_G):
                    pltpu.make_async_copy(
                        rows_v.at[j], out_hbm.at[row0 - _G + j], sem_st
                    ).wait()

            pltpu.sync_copy(idx_hbm.at[pl.ds(row0, _G)], idx_v)
            gathers = [
                pltpu.async_copy(tab_hbm.at[idx_v.at[j]], rows_v.at[j], sem_g)
                for j in range(_G)
            ]
            for j in range(_G):
                gathers[j].wait()
                pltpu.async_copy(rows_v.at[j], out_hbm.at[row0 + j], sem_st)
            return carry

        lax.fori_loop(0, _GROUPS, group, 0)
        last0 = base_row + (_GROUPS - 1) * _G
        for j in range(_G):
            pltpu.make_async_copy(
                rows_v.at[j], out_hbm.at[last0 + j], sem_st
            ).wait()

    run(idx_a_hbm, tab_a_hbm, out_a_hbm, idx_a_v, rows_a_v)
    run(idx_b_hbm, tab_b_hbm, out_b_hbm, idx_b_v, rows_b_v)


@jax.jit
def _lookup(dom_a_idx, dom_b_idx, table_a, table_b):
    mesh = plsc.VectorSubcoreMesh(core_axis_name="c", subcore_axis_name="s")
    k = pl.kernel(
        _body,
        out_type=(
            jax.ShapeDtypeStruct((_ROWS, _L, _DIM_A), jnp.float32),
            jax.ShapeDtypeStruct((_ROWS, _L, _DIM_B), jnp.float32),
        ),
        mesh=mesh,
        scratch_types=[
            pltpu.VMEM((_G, _L), jnp.int32),
            pltpu.VMEM((_G, _L), jnp.int32),
            pltpu.VMEM((_G, _L, _DIM_A), jnp.float32),
            pltpu.VMEM((_G, _L, _DIM_B), jnp.float32),
            pltpu.SemaphoreType.DMA,
            pltpu.SemaphoreType.DMA,
        ],
        compiler_params=pltpu.CompilerParams(use_tc_tiling_on_sc=False),
    )
    ia = dom_a_idx.reshape(_ROWS, _L)
    ib = dom_b_idx.reshape(_ROWS, _L)
    out_a, out_b = k(ia, ib, table_a, table_b)
    return (
        out_a.reshape(_BATCH, _HIST, _DIM_A),
        out_b.reshape(_BATCH, _HIST, _DIM_B),
    )


def kernel(dom_a_idx, dom_b_idx, table_a, table_b):
    return _lookup(dom_a_idx, dom_b_idx, table_a, table_b)
